# baseline (speedup 1.0000x reference)
import functools

import jax
import jax.numpy as jnp
from jax import lax
from jax.experimental import pallas as pl
from jax.experimental.pallas import tpu as pltpu
from jax.experimental.pallas import tpu_sc as plsc

N_FRAG = 320000
D = 128
N_GEN = 10000

NC = 2
NS = 16
FRAG_PER_TILE = N_FRAG // (NC * NS)
FRAG_PER_CORE = N_FRAG // NC
CHUNK = 80
N_CHUNK = FRAG_PER_TILE // CHUNK
NBUF = 4
SLACK = 2
GEN_SLICE = 624
GEN_REM_OFF = NS * GEN_SLICE
GEN_REM = N_GEN - GEN_REM_OFF
ZROWS = 16


def _sc_partials(embedding, idx32):
    mesh = plsc.VectorSubcoreMesh(core_axis_name="c", subcore_axis_name="s")

    @functools.partial(
        pl.kernel,
        out_type=jax.ShapeDtypeStruct((NC, N_GEN, D), jnp.float32),
        mesh=mesh,
        scratch_types=[
            pltpu.VMEM((NBUF, CHUNK, D), jnp.float32),
            pltpu.VMEM((NBUF, CHUNK), jnp.int32),
            pltpu.VMEM((ZROWS, D), jnp.float32),
            pltpu.VMEM_SHARED((N_GEN, D), jnp.float32),
        ] + [pltpu.SemaphoreType.DMA] * (3 * NBUF + 1),
    )
    def k(emb_hbm, idx_hbm, part_hbm, rows_v, idx_v, zbuf, acc, *sems):
        c = lax.axis_index("c")
        s = lax.axis_index("s")
        base = c * FRAG_PER_CORE + s * FRAG_PER_TILE
        rsems = sems[:NBUF]
        isems = sems[NBUF:2 * NBUF]
        ssems = sems[2 * NBUF:3 * NBUF]
        zsem = sems[3 * NBUF]

        def start_gather(j, b):
            off = base + j * CHUNK
            pltpu.async_copy(emb_hbm.at[pl.ds(off, CHUNK)], rows_v.at[b],
                             rsems[b])
            pltpu.async_copy(idx_hbm.at[pl.ds(off, CHUNK)], idx_v.at[b],
                             isems[b])

        def wait_gather(b):
            pltpu.make_async_copy(
                emb_hbm.at[pl.ds(0, CHUNK)], rows_v.at[b], rsems[b]).wait()
            pltpu.make_async_copy(
                idx_hbm.at[pl.ds(0, CHUNK)], idx_v.at[b], isems[b]).wait()

        def wait_scatter(b):
            pltpu.make_async_copy(
                rows_v.at[b], acc.at[idx_v.at[b]], ssems[b]).wait()

        for b in range(NBUF - SLACK):
            start_gather(b, b)

        def zrow(i, _):
            def zcol(j, _):
                zbuf[i, pl.ds(j * 16, 16)] = jnp.zeros((16,), jnp.float32)
                return 0
            return lax.fori_loop(0, D // 16, zcol, 0)
        lax.fori_loop(0, ZROWS, zrow, 0)

        def zcopy(z, _):
            pltpu.async_copy(
                zbuf, acc.at[pl.ds(s * GEN_SLICE + z * ZROWS, ZROWS)], zsem)
            return 0
        lax.fori_loop(0, GEN_SLICE // ZROWS, zcopy, 0)

        @pl.when(s == NS - 1)
        def _():
            pltpu.async_copy(zbuf, acc.at[pl.ds(GEN_REM_OFF, GEN_REM)], zsem)

        def zdrain(z, _):
            pltpu.make_async_copy(
                zbuf, acc.at[pl.ds(0, ZROWS)], zsem).wait()
            return 0
        lax.fori_loop(0, GEN_SLICE // ZROWS, zdrain, 0)

        @pl.when(s == NS - 1)
        def _():
            pltpu.make_async_copy(zbuf, acc.at[pl.ds(0, GEN_REM)], zsem).wait()

        plsc.subcore_barrier()

        def group_body(q, _):
            for b in range(NBUF):
                j = NBUF * q + b

                @pl.when(j < N_CHUNK)
                def _():
                    wait_gather(b)
                    pltpu.async_copy(rows_v.at[b], acc.at[idx_v.at[b]],
                                     ssems[b], add=True)
                    jj = j + (NBUF - SLACK)
                    bb = (b + NBUF - SLACK) % NBUF

                    @pl.when(jj < N_CHUNK)
                    def _():
                        @pl.when(jj >= NBUF)
                        def _():
                            wait_scatter(bb)
                        start_gather(jj, bb)
            return 0
        lax.fori_loop(0, (N_CHUNK + NBUF - 1) // NBUF, group_body, 0)

        for b in range(NBUF):
            wait_scatter(b)

        plsc.subcore_barrier()
        pltpu.sync_copy(
            acc.at[pl.ds(s * GEN_SLICE, GEN_SLICE)],
            part_hbm.at[c, pl.ds(s * GEN_SLICE, GEN_SLICE)])

        @pl.when(s == NS - 1)
        def _():
            pltpu.sync_copy(
                acc.at[pl.ds(GEN_REM_OFF, GEN_REM)],
                part_hbm.at[c, pl.ds(GEN_REM_OFF, GEN_REM)])

    return k(embedding, idx32)


def _combine_kernel(p_ref, o_ref):
    o_ref[...] = p_ref[0] + p_ref[1]


def _combine(partials):
    blk = 1000
    return pl.pallas_call(
        _combine_kernel,
        grid=(N_GEN // blk,),
        in_specs=[pl.BlockSpec((NC, blk, D), lambda i: (0, i, 0))],
        out_specs=pl.BlockSpec((blk, D), lambda i: (i, 0)),
        out_shape=jax.ShapeDtypeStruct((N_GEN, D), jnp.float32),
    )(partials)


def kernel(embedding, local_gene_ix, n_genes):
    idx32 = local_gene_ix.astype(jnp.int32)
    partials = _sc_partials(embedding, idx32)
    return _combine(partials)


# speedup vs baseline: 1.0869x; 1.0869x over previous
import functools

import jax
import jax.numpy as jnp
from jax import lax
from jax.experimental import pallas as pl
from jax.experimental.pallas import tpu as pltpu
from jax.experimental.pallas import tpu_sc as plsc

N_FRAG = 320000
D = 128
N_GEN = 10000

NC = 2
NS = 16
FRAG_PER_TILE = N_FRAG // (NC * NS)
FRAG_PER_CORE = N_FRAG // NC
CHUNK = 80
N_CHUNK = FRAG_PER_TILE // CHUNK
NBUF = 4
SLACK = 1
GEN_SLICE = 624
GEN_REM_OFF = NS * GEN_SLICE
GEN_REM = N_GEN - GEN_REM_OFF
ZROWS = 16


def _sc_partials(embedding, idx32):
    mesh = plsc.VectorSubcoreMesh(core_axis_name="c", subcore_axis_name="s")

    @functools.partial(
        pl.kernel,
        out_type=jax.ShapeDtypeStruct((NC, N_GEN, D), jnp.float32),
        mesh=mesh,
        scratch_types=[
            pltpu.VMEM((NBUF, CHUNK, D), jnp.float32),
            pltpu.VMEM((NBUF, CHUNK), jnp.int32),
            pltpu.VMEM((ZROWS, D), jnp.float32),
            pltpu.VMEM_SHARED((N_GEN, D), jnp.float32),
        ] + [pltpu.SemaphoreType.DMA] * (3 * NBUF + 1),
    )
    def k(emb_hbm, idx_hbm, part_hbm, rows_v, idx_v, zbuf, acc, *sems):
        c = lax.axis_index("c")
        s = lax.axis_index("s")
        base = c * FRAG_PER_CORE + s * FRAG_PER_TILE
        rsems = sems[:NBUF]
        isems = sems[NBUF:2 * NBUF]
        ssems = sems[2 * NBUF:3 * NBUF]
        zsem = sems[3 * NBUF]

        def start_gather(j, b):
            off = base + j * CHUNK
            pltpu.async_copy(emb_hbm.at[pl.ds(off, CHUNK)], rows_v.at[b],
                             rsems[b])
            pltpu.async_copy(idx_hbm.at[pl.ds(off, CHUNK)], idx_v.at[b],
                             isems[b])

        def wait_gather(b):
            pltpu.make_async_copy(
                emb_hbm.at[pl.ds(0, CHUNK)], rows_v.at[b], rsems[b]).wait()
            pltpu.make_async_copy(
                idx_hbm.at[pl.ds(0, CHUNK)], idx_v.at[b], isems[b]).wait()

        def wait_scatter(b):
            pltpu.make_async_copy(
                rows_v.at[b], acc.at[idx_v.at[b]], ssems[b]).wait()

        for b in range(NBUF - SLACK):
            start_gather(b, b)

        def zrow(i, _):
            def zcol(j, _):
                zbuf[i, pl.ds(j * 16, 16)] = jnp.zeros((16,), jnp.float32)
                return 0
            return lax.fori_loop(0, D // 16, zcol, 0)
        lax.fori_loop(0, ZROWS, zrow, 0)

        def zcopy(z, _):
            pltpu.async_copy(
                zbuf, acc.at[pl.ds(s * GEN_SLICE + z * ZROWS, ZROWS)], zsem)
            return 0
        lax.fori_loop(0, GEN_SLICE // ZROWS, zcopy, 0)

        @pl.when(s == NS - 1)
        def _():
            pltpu.async_copy(zbuf, acc.at[pl.ds(GEN_REM_OFF, GEN_REM)], zsem)

        def zdrain(z, _):
            pltpu.make_async_copy(
                zbuf, acc.at[pl.ds(0, ZROWS)], zsem).wait()
            return 0
        lax.fori_loop(0, GEN_SLICE // ZROWS, zdrain, 0)

        @pl.when(s == NS - 1)
        def _():
            pltpu.make_async_copy(zbuf, acc.at[pl.ds(0, GEN_REM)], zsem).wait()

        plsc.subcore_barrier()

        def group_body(q, _):
            for b in range(NBUF):
                j = NBUF * q + b

                @pl.when(j < N_CHUNK)
                def _():
                    wait_gather(b)
                    pltpu.async_copy(rows_v.at[b], acc.at[idx_v.at[b]],
                                     ssems[b], add=True)
                    jj = j + (NBUF - SLACK)
                    bb = (b + NBUF - SLACK) % NBUF

                    @pl.when(jj < N_CHUNK)
                    def _():
                        @pl.when(jj >= NBUF)
                        def _():
                            wait_scatter(bb)
                        start_gather(jj, bb)
            return 0
        lax.fori_loop(0, (N_CHUNK + NBUF - 1) // NBUF, group_body, 0)

        for b in range(NBUF):
            wait_scatter(b)

        plsc.subcore_barrier()
        pltpu.sync_copy(
            acc.at[pl.ds(s * GEN_SLICE, GEN_SLICE)],
            part_hbm.at[c, pl.ds(s * GEN_SLICE, GEN_SLICE)])

        @pl.when(s == NS - 1)
        def _():
            pltpu.sync_copy(
                acc.at[pl.ds(GEN_REM_OFF, GEN_REM)],
                part_hbm.at[c, pl.ds(GEN_REM_OFF, GEN_REM)])

    return k(embedding, idx32)


def _combine_kernel(p_ref, o_ref):
    o_ref[...] = p_ref[0] + p_ref[1]


def _combine(partials):
    blk = 1000
    return pl.pallas_call(
        _combine_kernel,
        grid=(N_GEN // blk,),
        in_specs=[pl.BlockSpec((NC, blk, D), lambda i: (0, i, 0))],
        out_specs=pl.BlockSpec((blk, D), lambda i: (i, 0)),
        out_shape=jax.ShapeDtypeStruct((N_GEN, D), jnp.float32),
    )(partials)


def kernel(embedding, local_gene_ix, n_genes):
    idx32 = local_gene_ix.astype(jnp.int32)
    partials = _sc_partials(embedding, idx32)
    return _combine(partials)


# speedup vs baseline: 1.1606x; 1.0677x over previous
import functools

import jax
import jax.numpy as jnp
from jax import lax
from jax.experimental import pallas as pl
from jax.experimental.pallas import tpu as pltpu
from jax.experimental.pallas import tpu_sc as plsc

N_FRAG = 320000
D = 128
N_GEN = 10000

NC = 2
NS = 16
FRAG_PER_TILE = N_FRAG // (NC * NS)
FRAG_PER_CORE = N_FRAG // NC
CHUNK = 40
N_CHUNK = FRAG_PER_TILE // CHUNK
NBUF = 8
SLACK = 2
GEN_SLICE = 624
GEN_REM_OFF = NS * GEN_SLICE
GEN_REM = N_GEN - GEN_REM_OFF
ZROWS = 16


def _sc_partials(embedding, idx32):
    mesh = plsc.VectorSubcoreMesh(core_axis_name="c", subcore_axis_name="s")

    @functools.partial(
        pl.kernel,
        out_type=jax.ShapeDtypeStruct((NC, N_GEN, D), jnp.float32),
        mesh=mesh,
        scratch_types=[
            pltpu.VMEM((NBUF, CHUNK, D), jnp.float32),
            pltpu.VMEM((NBUF, CHUNK), jnp.int32),
            pltpu.VMEM((ZROWS, D), jnp.float32),
            pltpu.VMEM_SHARED((N_GEN, D), jnp.float32),
        ] + [pltpu.SemaphoreType.DMA] * (3 * NBUF + 1),
    )
    def k(emb_hbm, idx_hbm, part_hbm, rows_v, idx_v, zbuf, acc, *sems):
        c = lax.axis_index("c")
        s = lax.axis_index("s")
        base = c * FRAG_PER_CORE + s * FRAG_PER_TILE
        rsems = sems[:NBUF]
        isems = sems[NBUF:2 * NBUF]
        ssems = sems[2 * NBUF:3 * NBUF]
        zsem = sems[3 * NBUF]

        def start_gather(j, b):
            off = base + j * CHUNK
            pltpu.async_copy(emb_hbm.at[pl.ds(off, CHUNK)], rows_v.at[b],
                             rsems[b])
            pltpu.async_copy(idx_hbm.at[pl.ds(off, CHUNK)], idx_v.at[b],
                             isems[b])

        def wait_gather(b):
            pltpu.make_async_copy(
                emb_hbm.at[pl.ds(0, CHUNK)], rows_v.at[b], rsems[b]).wait()
            pltpu.make_async_copy(
                idx_hbm.at[pl.ds(0, CHUNK)], idx_v.at[b], isems[b]).wait()

        def wait_scatter(b):
            pltpu.make_async_copy(
                rows_v.at[b], acc.at[idx_v.at[b]], ssems[b]).wait()

        for b in range(NBUF - SLACK):
            start_gather(b, b)

        def zrow(i, _):
            def zcol(j, _):
                zbuf[i, pl.ds(j * 16, 16)] = jnp.zeros((16,), jnp.float32)
                return 0
            return lax.fori_loop(0, D // 16, zcol, 0)
        lax.fori_loop(0, ZROWS, zrow, 0)

        def zcopy(z, _):
            pltpu.async_copy(
                zbuf, acc.at[pl.ds(s * GEN_SLICE + z * ZROWS, ZROWS)], zsem)
            return 0
        lax.fori_loop(0, GEN_SLICE // ZROWS, zcopy, 0)

        @pl.when(s == NS - 1)
        def _():
            pltpu.async_copy(zbuf, acc.at[pl.ds(GEN_REM_OFF, GEN_REM)], zsem)

        def zdrain(z, _):
            pltpu.make_async_copy(
                zbuf, acc.at[pl.ds(0, ZROWS)], zsem).wait()
            return 0
        lax.fori_loop(0, GEN_SLICE // ZROWS, zdrain, 0)

        @pl.when(s == NS - 1)
        def _():
            pltpu.make_async_copy(zbuf, acc.at[pl.ds(0, GEN_REM)], zsem).wait()

        plsc.subcore_barrier()

        def group_body(q, _):
            for b in range(NBUF):
                j = NBUF * q + b

                @pl.when(j < N_CHUNK)
                def _():
                    wait_gather(b)
                    pltpu.async_copy(rows_v.at[b], acc.at[idx_v.at[b]],
                                     ssems[b], add=True)
                    jj = j + (NBUF - SLACK)
                    bb = (b + NBUF - SLACK) % NBUF

                    @pl.when(jj < N_CHUNK)
                    def _():
                        @pl.when(jj >= NBUF)
                        def _():
                            wait_scatter(bb)
                        start_gather(jj, bb)
            return 0
        lax.fori_loop(0, (N_CHUNK + NBUF - 1) // NBUF, group_body, 0)

        for b in range(NBUF):
            wait_scatter(b)

        plsc.subcore_barrier()
        pltpu.sync_copy(
            acc.at[pl.ds(s * GEN_SLICE, GEN_SLICE)],
            part_hbm.at[c, pl.ds(s * GEN_SLICE, GEN_SLICE)])

        @pl.when(s == NS - 1)
        def _():
            pltpu.sync_copy(
                acc.at[pl.ds(GEN_REM_OFF, GEN_REM)],
                part_hbm.at[c, pl.ds(GEN_REM_OFF, GEN_REM)])

    return k(embedding, idx32)


def _combine_kernel(p_ref, o_ref):
    o_ref[...] = p_ref[0] + p_ref[1]


def _combine(partials):
    blk = 1000
    return pl.pallas_call(
        _combine_kernel,
        grid=(N_GEN // blk,),
        in_specs=[pl.BlockSpec((NC, blk, D), lambda i: (0, i, 0))],
        out_specs=pl.BlockSpec((blk, D), lambda i: (i, 0)),
        out_shape=jax.ShapeDtypeStruct((N_GEN, D), jnp.float32),
    )(partials)


def kernel(embedding, local_gene_ix, n_genes):
    idx32 = local_gene_ix.astype(jnp.int32)
    partials = _sc_partials(embedding, idx32)
    return _combine(partials)


# speedup vs baseline: 1.1780x; 1.0150x over previous
import functools

import jax
import jax.numpy as jnp
from jax import lax
from jax.experimental import pallas as pl
from jax.experimental.pallas import tpu as pltpu
from jax.experimental.pallas import tpu_sc as plsc

N_FRAG = 320000
D = 128
N_GEN = 10000

NC = 2
NS = 16
GENE_SPLIT = 5008
ACC_ROWS = N_GEN + 8
W = 40
N_WIN = N_FRAG // W
NBUF = 8
SLACK = 2
N_PROBE = 5
GEN_SLICE = 312
SC0_REM_OFF = NS * GEN_SLICE
SC0_REM = GENE_SPLIT - SC0_REM_OFF
ZROWS = 48
ZN_FULL = 13
ZREM = 16


def _sc_segment_sum(embedding, idx32):
    mesh = plsc.VectorSubcoreMesh(core_axis_name="c", subcore_axis_name="s")

    @functools.partial(
        pl.kernel,
        out_type=jax.ShapeDtypeStruct((N_GEN, D), jnp.float32),
        mesh=mesh,
        scratch_types=[
            pltpu.VMEM((NBUF, W, D), jnp.float32),
            pltpu.VMEM((NBUF, W), jnp.int32),
            pltpu.VMEM((ZROWS, D), jnp.float32),
            pltpu.VMEM((128,), jnp.int32),
            pltpu.SMEM((128,), jnp.int32),
            pltpu.VMEM_SHARED((NS, 128), jnp.int32),
            pltpu.VMEM_SHARED((ACC_ROWS, D), jnp.float32),
        ] + [pltpu.SemaphoreType.DMA] * (3 * NBUF + 1),
    )
    def k(emb_hbm, idx_hbm, out_hbm, rows_v, idx_v, zbuf, probe_v, probe_s,
          probe_sh, acc, *sems):
        c = lax.axis_index("c")
        s = lax.axis_index("s")
        rsems = sems[:NBUF]
        isems = sems[NBUF:2 * NBUF]
        ssems = sems[2 * NBUF:3 * NBUF]
        psem = sems[3 * NBUF]

        def zrow(i, _):
            def zcol(jj, _):
                zbuf[i, pl.ds(jj * 16, 16)] = jnp.zeros((16,), jnp.float32)
                return 0
            return lax.fori_loop(0, D // 16, zcol, 0)
        lax.fori_loop(0, ZROWS, zrow, 0)

        def zcopy(z, _):
            pltpu.sync_copy(
                zbuf, acc.at[pl.ds(s * (ZN_FULL * ZROWS) + z * ZROWS, ZROWS)])
            return 0
        lax.fori_loop(0, ZN_FULL, zcopy, 0)

        @pl.when(s == NS - 1)
        def _():
            pltpu.sync_copy(zbuf.at[pl.ds(0, ZREM)],
                            acc.at[pl.ds(NS * ZN_FULL * ZROWS, ZREM)])

        def probe(_, carry):
            lo_b, hi_b = carry
            span_b = hi_b - lo_b
            for i in range(16):
                m_i = jnp.minimum(
                    lo_b + (span_b * (i + 1)) // 16, N_FRAG - 1)
                w_i = pl.multiple_of(
                    jnp.minimum(m_i & -8, N_FRAG - 8), 8)
                pltpu.async_copy(idx_hbm.at[pl.ds(w_i, 8)],
                                 probe_v.at[pl.ds(i * 8, 8)], psem)
            for i in range(16):
                pltpu.make_async_copy(idx_hbm.at[pl.ds(0, 8)],
                                      probe_v.at[pl.ds(i * 8, 8)],
                                      psem).wait()
            pltpu.sync_copy(probe_v, probe_sh.at[s])
            pltpu.sync_copy(probe_sh.at[s], probe_s)
            lo_n, hi_n = lo_b, hi_b
            for i in range(15, -1, -1):
                m_i = lo_b + (span_b * (i + 1)) // 16
                m_c = jnp.minimum(m_i, N_FRAG - 1)
                w_i = jnp.minimum(m_c & -8, N_FRAG - 8)
                v_i = probe_s[i * 8 + (m_c - w_i)]
                ge_i = v_i >= GENE_SPLIT
                hi_n = jnp.where(ge_i, m_i, hi_n)
            for i in range(16):
                m_i = lo_b + (span_b * (i + 1)) // 16
                m_c = jnp.minimum(m_i, N_FRAG - 1)
                w_i = jnp.minimum(m_c & -8, N_FRAG - 8)
                v_i = probe_s[i * 8 + (m_c - w_i)]
                lo_n = jnp.where(v_i >= GENE_SPLIT, lo_n, m_i + 1)
            return jnp.minimum(lo_n, hi_n), hi_n
        split, _ = lax.fori_loop(0, N_PROBE, probe, (0, N_FRAG))

        w0 = jnp.where(c == 0, 0, split // W)
        w1 = jnp.where(c == 0, (split + W - 1) // W, N_WIN)
        nw = w1 - w0
        ws = w0 + (nw * s) // NS
        nwin = w0 + (nw * (s + 1)) // NS - ws

        def start_gather(t, b):
            off = (ws + t) * W
            pltpu.async_copy(emb_hbm.at[pl.ds(off, W)], rows_v.at[b],
                             rsems[b])
            pltpu.async_copy(idx_hbm.at[pl.ds(off, W)], idx_v.at[b],
                             isems[b])

        def wait_gather(b):
            pltpu.make_async_copy(
                emb_hbm.at[pl.ds(0, W)], rows_v.at[b], rsems[b]).wait()
            pltpu.make_async_copy(
                idx_hbm.at[pl.ds(0, W)], idx_v.at[b], isems[b]).wait()

        def wait_scatter(b):
            pltpu.make_async_copy(
                rows_v.at[b], acc.at[idx_v.at[b]], ssems[b]).wait()

        for b in range(NBUF - SLACK):
            @pl.when(b < nwin)
            def _():
                start_gather(b, b)

        plsc.subcore_barrier()

        def group_body(q, _):
            for b in range(NBUF):
                t = NBUF * q + b

                @pl.when(t < nwin)
                def _():
                    wait_gather(b)
                    pltpu.async_copy(rows_v.at[b], acc.at[idx_v.at[b]],
                                     ssems[b], add=True)
                    tt = t + (NBUF - SLACK)
                    bb = (b + NBUF - SLACK) % NBUF

                    @pl.when(tt < nwin)
                    def _():
                        @pl.when(tt >= NBUF)
                        def _():
                            wait_scatter(bb)
                        start_gather(tt, bb)
            return 0
        lax.fori_loop(0, (nwin + NBUF - 1) // NBUF, group_body, 0)

        for b in range(NBUF):
            @pl.when(b < nwin)
            def _():
                wait_scatter(b)

        plsc.subcore_barrier()

        pltpu.sync_copy(
            acc.at[pl.ds(c * GENE_SPLIT + s * GEN_SLICE, GEN_SLICE)],
            out_hbm.at[pl.ds(c * GENE_SPLIT + s * GEN_SLICE, GEN_SLICE)])

        @pl.when((s == NS - 1) & (c == 0))
        def _():
            pltpu.sync_copy(
                acc.at[pl.ds(SC0_REM_OFF, SC0_REM)],
                out_hbm.at[pl.ds(SC0_REM_OFF, SC0_REM)])

    return k(embedding, idx32)


def kernel(embedding, local_gene_ix, n_genes):
    idx32 = local_gene_ix.astype(jnp.int32)
    return _sc_segment_sum(embedding, idx32)
